# baseline (speedup 1.0000x reference)
import functools

import jax
import jax.numpy as jnp
from jax import lax
from jax.experimental import pallas as pl
from jax.experimental.pallas import tpu as pltpu
from jax.experimental.pallas import tpu_sc as plsc

B = 32
N = 8192
KSEL = 256
L = 16
NCHUNK = N // L
NC = 2
NS = 16

SEG = N + 32
BUFLEN = 2 * SEG
HI_BASE = 0
LO_BASE = SEG

BITS_HI = 0x7F800001

PA = 0x3FF33333
PB = 0x401CCCCD


def _partition_pass(src, dst, state, lanes):
    lo, hi, goal, off, n = state
    done = (hi - lo) <= 1
    small = jnp.logical_not(done) & (n <= L)
    active = jnp.logical_not(done) & jnp.logical_not(small)
    mid = lo + ((hi - lo) >> 1)
    nch = jnp.where(active, (n + L - 1) // L, 0)

    def body(i, carry):
        phi, plo = carry
        base = i * L
        bits = src[pl.ds(off + base, L)]
        valid = lanes < (n - base)
        mhi = valid & (bits >= mid)
        mlo = valid & (bits < mid)
        chi = jnp.sum(mhi.astype(jnp.int32))
        clo = jnp.sum(mlo.astype(jnp.int32))
        plsc.store_compressed(dst.at[pl.ds(HI_BASE + phi, L)], bits, mask=mhi)
        plsc.store_compressed(dst.at[pl.ds(LO_BASE + plo, L)], bits, mask=mlo)
        return (phi + chi, plo + clo)

    chi_t, clo_t = lax.fori_loop(0, nch, body, (jnp.int32(0), jnp.int32(0)))

    v = src[pl.ds(off, L)]
    sk = plsc.sort_key_val(v, v, mask=lanes < n, descending=True)[0]
    pick = jnp.sum(jnp.where(lanes == goal - 1, sk, jnp.int32(0)))

    take_hi = chi_t >= goal
    lo2 = jnp.where(done, lo,
                    jnp.where(small, pick, jnp.where(take_hi, mid, lo)))
    hi2 = jnp.where(done, hi,
                    jnp.where(small, pick + 1, jnp.where(take_hi, hi, mid)))
    keep = done | small
    goal2 = jnp.where(keep | take_hi, goal, goal - chi_t)
    off2 = jnp.where(keep, off, jnp.where(take_hi, HI_BASE, LO_BASE))
    n2 = jnp.where(keep, n, jnp.where(take_hi, chi_t, clo_t))
    return (lo2, hi2, goal2, off2, n2)


def _kwta_body(x_hbm, out_hbm, row_v, b0, b1,
               si0, si1, si2, si3, so0, so1, so2, so3):
    wid = lax.axis_index("s") * NC + lax.axis_index("c")

    SECN = 4
    SECE = N // SECN
    in_sems = (si0, si1, si2, si3)
    out_sems = (so0, so1, so2, so3)
    in_copies = [
        pltpu.make_async_copy(
            x_hbm.at[wid, pl.ds(s * SECE, SECE)],
            row_v.at[pl.ds(s * SECE, SECE)], in_sems[s])
        for s in range(SECN)
    ]
    for c in in_copies:
        c.start()

    lanes = lax.iota(jnp.int32, L)

    def p1_body(i, carry):
        qm, cv = carry
        xc = row_v[pl.ds(i * L, L)]
        bits = plsc.bitcast(xc, jnp.int32) & jnp.int32(0x7FFFFFFF)
        geb = bits >= PB
        m_mid = (bits >= PA) & jnp.logical_not(geb)
        c_mid = jnp.sum(m_mid.astype(jnp.int32))
        plsc.store_compressed(b0.at[pl.ds(qm, L)], bits, mask=m_mid)
        return (qm + c_mid, cv + geb.astype(jnp.int32))

    zv = jnp.zeros((L,), jnp.int32)
    carry = (jnp.int32(0), zv)
    for s in range(SECN):
        in_copies[s].wait()
        carry = plsc.parallel_loop(
            s * (NCHUNK // SECN), (s + 1) * (NCHUNK // SECN),
            carry=carry, unroll=8)(p1_body)
    n_mid, cv = carry
    c_hi = jnp.sum(cv)
    hi_cap = jnp.int32(BITS_HI)

    in_band = (c_hi < KSEL) & (c_hi + n_mid >= KSEL)

    s_lo = jnp.where(c_hi >= KSEL, jnp.int32(PB), jnp.int32(0))
    s_hi = jnp.where(c_hi >= KSEL, jnp.int32(BITS_HI), jnp.int32(PA))

    def fb(_):
        def fbody(i, q):
            xc = row_v[pl.ds(i * L, L)]
            bits = plsc.bitcast(xc, jnp.int32) & jnp.int32(0x7FFFFFFF)
            m = (bits >= s_lo) & (bits < s_hi)
            c = jnp.sum(m.astype(jnp.int32))
            plsc.store_compressed(b0.at[pl.ds(SEG + q, L)], bits, mask=m)
            return q + c

        return lax.fori_loop(0, NCHUNK, fbody, jnp.int32(0))

    nfb = lax.cond(in_band, lambda _: jnp.int32(0), fb, 0)

    state = (
        jnp.where(in_band, jnp.int32(PA), s_lo),
        jnp.where(in_band, jnp.int32(PB), s_hi),
        jnp.where(c_hi >= KSEL, jnp.int32(KSEL),
                  jnp.where(in_band, KSEL - c_hi, KSEL - c_hi - n_mid)),
        jnp.where(in_band, jnp.int32(HI_BASE), jnp.int32(SEG)),
        jnp.where(in_band, n_mid, nfb),
    )

    def dbl(st):
        st = _partition_pass(b0, b1, st, lanes)
        st = _partition_pass(b1, b0, st, lanes)
        return st

    state = lax.while_loop(lambda st: (st[1] - st[0]) > 1, dbl, state)
    thresh = state[0]

    def fin_body(i):
        xc = row_v[pl.ds(i * L, L)]
        bits = plsc.bitcast(xc, jnp.int32) & jnp.int32(0x7FFFFFFF)
        row_v[pl.ds(i * L, L)] = jnp.where(bits >= thresh, xc, jnp.float32(0.0))

    out_copies = []
    for s in range(SECN):
        plsc.parallel_loop(
            s * (NCHUNK // SECN), (s + 1) * (NCHUNK // SECN),
            unroll=8)(fin_body)
        c = pltpu.make_async_copy(
            row_v.at[pl.ds(s * SECE, SECE)],
            out_hbm.at[wid, pl.ds(s * SECE, SECE)], out_sems[s])
        c.start()
        out_copies.append(c)
    for c in out_copies:
        c.wait()


@functools.cache
def _kwta():
    mesh = plsc.VectorSubcoreMesh(core_axis_name="c", subcore_axis_name="s",
                                  num_cores=NC, num_subcores=NS)
    return pl.kernel(
        _kwta_body,
        out_type=jax.ShapeDtypeStruct((B, N), jnp.float32),
        mesh=mesh,
        scratch_types=[
            pltpu.VMEM((N,), jnp.float32),
            pltpu.VMEM((BUFLEN,), jnp.int32),
            pltpu.VMEM((BUFLEN,), jnp.int32),
        ] + [pltpu.SemaphoreType.DMA] * 8,
        compiler_params=pltpu.CompilerParams(needs_layout_passes=False),
    )


@jax.jit
def kernel(input_tensor):
    return _kwta()(input_tensor)


# speedup vs baseline: 1.0300x; 1.0300x over previous
import functools

import jax
import jax.numpy as jnp
from jax import lax
from jax.experimental import pallas as pl
from jax.experimental.pallas import tpu as pltpu
from jax.experimental.pallas import tpu_sc as plsc

B = 32
N = 8192
KSEL = 256
L = 16
NCHUNK = N // L
NC = 2
NS = 16

SEG = N + 32
BUFLEN = 2 * SEG
HI_BASE = 0
LO_BASE = SEG

BITS_HI = 0x7F800001

PA = 0x3FF33333
PB = 0x401CCCCD


def _partition_pass(src, dst, state, lanes):
    lo, hi, goal, off, n = state
    done = (hi - lo) <= 1
    small = jnp.logical_not(done) & (n <= L)
    active = jnp.logical_not(done) & jnp.logical_not(small)
    mid = lo + ((hi - lo) >> 1)
    nch = jnp.where(active, (n + L - 1) // L, 0)

    def body(i, carry):
        phi, plo = carry
        base = i * L
        bits = src[pl.ds(off + base, L)]
        valid = lanes < (n - base)
        mhi = valid & (bits >= mid)
        mlo = valid & (bits < mid)
        chi = jnp.sum(mhi.astype(jnp.int32))
        clo = jnp.sum(mlo.astype(jnp.int32))
        plsc.store_compressed(dst.at[pl.ds(HI_BASE + phi, L)], bits, mask=mhi)
        plsc.store_compressed(dst.at[pl.ds(LO_BASE + plo, L)], bits, mask=mlo)
        return (phi + chi, plo + clo)

    chi_t, clo_t = lax.fori_loop(0, nch, body, (jnp.int32(0), jnp.int32(0)))

    v = src[pl.ds(off, L)]
    sk = plsc.sort_key_val(v, v, mask=lanes < n, descending=True)[0]
    pick = jnp.sum(jnp.where(lanes == goal - 1, sk, jnp.int32(0)))

    take_hi = chi_t >= goal
    lo2 = jnp.where(done, lo,
                    jnp.where(small, pick, jnp.where(take_hi, mid, lo)))
    hi2 = jnp.where(done, hi,
                    jnp.where(small, pick + 1, jnp.where(take_hi, hi, mid)))
    keep = done | small
    goal2 = jnp.where(keep | take_hi, goal, goal - chi_t)
    off2 = jnp.where(keep, off, jnp.where(take_hi, HI_BASE, LO_BASE))
    n2 = jnp.where(keep, n, jnp.where(take_hi, chi_t, clo_t))
    return (lo2, hi2, goal2, off2, n2)


def _kwta_body(x_hbm, out_hbm, row_v, b0, b1,
               si0, si1, si2, si3, so0, so1, so2, so3):
    wid = lax.axis_index("s") * NC + lax.axis_index("c")

    SECN = 2
    SECE = N // SECN
    in_sems = (si0, si1)
    out_sems = (so0, so1)
    in_copies = [
        pltpu.make_async_copy(
            x_hbm.at[wid, pl.ds(s * SECE, SECE)],
            row_v.at[pl.ds(s * SECE, SECE)], in_sems[s])
        for s in range(SECN)
    ]
    for c in in_copies:
        c.start()

    lanes = lax.iota(jnp.int32, L)

    def p1_body(i, carry):
        qm, cv = carry
        xc = row_v[pl.ds(i * L, L)]
        bits = plsc.bitcast(xc, jnp.int32) & jnp.int32(0x7FFFFFFF)
        geb = bits >= PB
        m_mid = (bits >= PA) & jnp.logical_not(geb)
        c_mid = jnp.sum(m_mid.astype(jnp.int32))
        plsc.store_compressed(b0.at[pl.ds(qm, L)], bits, mask=m_mid)
        return (qm + c_mid, cv + geb.astype(jnp.int32))

    zv = jnp.zeros((L,), jnp.int32)
    carry = (jnp.int32(0), zv)
    for s in range(SECN):
        in_copies[s].wait()
        carry = plsc.parallel_loop(
            s * (NCHUNK // SECN), (s + 1) * (NCHUNK // SECN),
            carry=carry, unroll=8)(p1_body)
    n_mid, cv = carry
    c_hi = jnp.sum(cv)
    hi_cap = jnp.int32(BITS_HI)

    in_band = (c_hi < KSEL) & (c_hi + n_mid >= KSEL)

    s_lo = jnp.where(c_hi >= KSEL, jnp.int32(PB), jnp.int32(0))
    s_hi = jnp.where(c_hi >= KSEL, jnp.int32(BITS_HI), jnp.int32(PA))

    def fb(_):
        def fbody(i, q):
            xc = row_v[pl.ds(i * L, L)]
            bits = plsc.bitcast(xc, jnp.int32) & jnp.int32(0x7FFFFFFF)
            m = (bits >= s_lo) & (bits < s_hi)
            c = jnp.sum(m.astype(jnp.int32))
            plsc.store_compressed(b0.at[pl.ds(SEG + q, L)], bits, mask=m)
            return q + c

        return lax.fori_loop(0, NCHUNK, fbody, jnp.int32(0))

    nfb = lax.cond(in_band, lambda _: jnp.int32(0), fb, 0)

    state = (
        jnp.where(in_band, jnp.int32(PA), s_lo),
        jnp.where(in_band, jnp.int32(PB), s_hi),
        jnp.where(c_hi >= KSEL, jnp.int32(KSEL),
                  jnp.where(in_band, KSEL - c_hi, KSEL - c_hi - n_mid)),
        jnp.where(in_band, jnp.int32(HI_BASE), jnp.int32(SEG)),
        jnp.where(in_band, n_mid, nfb),
    )

    def dbl(st):
        st = _partition_pass(b0, b1, st, lanes)
        st = _partition_pass(b1, b0, st, lanes)
        return st

    state = lax.while_loop(lambda st: (st[1] - st[0]) > 1, dbl, state)
    thresh = state[0]

    def fin_body(i):
        xc = row_v[pl.ds(i * L, L)]
        bits = plsc.bitcast(xc, jnp.int32) & jnp.int32(0x7FFFFFFF)
        row_v[pl.ds(i * L, L)] = jnp.where(bits >= thresh, xc, jnp.float32(0.0))

    out_copies = []
    for s in range(SECN):
        plsc.parallel_loop(
            s * (NCHUNK // SECN), (s + 1) * (NCHUNK // SECN),
            unroll=8)(fin_body)
        c = pltpu.make_async_copy(
            row_v.at[pl.ds(s * SECE, SECE)],
            out_hbm.at[wid, pl.ds(s * SECE, SECE)], out_sems[s])
        c.start()
        out_copies.append(c)
    for c in out_copies:
        c.wait()


@functools.cache
def _kwta():
    mesh = plsc.VectorSubcoreMesh(core_axis_name="c", subcore_axis_name="s",
                                  num_cores=NC, num_subcores=NS)
    return pl.kernel(
        _kwta_body,
        out_type=jax.ShapeDtypeStruct((B, N), jnp.float32),
        mesh=mesh,
        scratch_types=[
            pltpu.VMEM((N,), jnp.float32),
            pltpu.VMEM((BUFLEN,), jnp.int32),
            pltpu.VMEM((BUFLEN,), jnp.int32),
        ] + [pltpu.SemaphoreType.DMA] * 8,
        compiler_params=pltpu.CompilerParams(needs_layout_passes=False),
    )


@jax.jit
def kernel(input_tensor):
    return _kwta()(input_tensor)
